# baseline (speedup 1.0000x reference)
import jax
import jax.numpy as jnp
from jax.experimental import pallas as pl


def _mask_body(s_ref, m_ref, o_ref):
    s = s_ref[...]
    o_ref[...] = jnp.where(m_ref[...] > 0, s, jnp.full_like(s, -1e9))


def kernel(scores, k):
    _, indices = jax.lax.top_k(scores, 256)
    b = scores.shape[0]
    row_ids = jnp.arange(b, dtype=indices.dtype)[:, None]
    mask = jnp.zeros(scores.shape, scores.dtype).at[row_ids, indices].set(1.0)
    masked = pl.pallas_call(
        _mask_body,
        out_shape=jax.ShapeDtypeStruct(scores.shape, scores.dtype),
        grid=(16,),
        in_specs=[
            pl.BlockSpec((8, 32768), lambda i: (i, 0)),
            pl.BlockSpec((8, 32768), lambda i: (i, 0)),
        ],
        out_specs=pl.BlockSpec((8, 32768), lambda i: (i, 0)),
    )(scores, mask)
    return masked, indices


# speedup vs baseline: 6.0156x; 6.0156x over previous
import jax
import jax.numpy as jnp
import numpy as np
from jax import lax
from jax.experimental import pallas as pl
from jax.experimental.pallas import tpu as pltpu
from jax.experimental.pallas import tpu_sc as plsc

B = 128
N = 32768
K = 256
NV = N // 16
NEG = np.float32(-1e9)
MIN32 = np.int32(-(2**31))


def _key_of(x):
    u = plsc.bitcast(x, jnp.int32)
    m = lax.shift_right_arithmetic(u, 31)
    return plsc.bitcast(u ^ (m | MIN32), jnp.uint32)


def _body(scores_hbm, masked_hbm, idx_hbm,
          row_v, cand_v, hist_v, tot_v, selk_v, seli_v, oidx_v):
    lane = lax.iota(jnp.int32, 16)
    zeros16 = lane ^ lane
    ones16 = zeros16 + np.int32(1)
    wid = lax.axis_index("s") * 2 + lax.axis_index("c")

    def zero_hist():
        def z(i, c):
            hist_v[pl.ds(i * 16, 16)] = zeros16
            return c
        lax.fori_loop(0, 256, z, 0)

    def select_level(need):
        def tot_g(g, c):
            def tot_c(ci, acc):
                rot = (lane + ci) & 15
                return acc + plsc.load_gather(
                    hist_v, [(g * 16 + lane) * 16 + rot])
            tot_v[pl.ds(g * 16, 16)] = lax.fori_loop(0, 16, tot_c, zeros16)
            return c
        lax.fori_loop(0, 16, tot_g, 0)

        def sel_g(i, carry):
            above, dplus, gcnt = carry
            g = 15 - i
            v = tot_v[pl.ds(g * 16, 16)]
            sufi = jnp.flip(jnp.cumsum(jnp.flip(v)))
            cgt = above + sufi - v
            msel = (cgt < need) & ((cgt + v) >= need)
            dplus = dplus + jnp.sum(jnp.where(msel, g * 16 + lane + 1, 0))
            gcnt = gcnt + jnp.sum(jnp.where(msel, cgt, 0))
            return above + jnp.sum(v), dplus, gcnt
        _, dplus, gcnt = lax.fori_loop(
            0, 16, sel_g, (np.int32(0), np.int32(0), np.int32(0)))
        return dplus - 1, gcnt

    def do_row(r):
        pltpu.sync_copy(scores_hbm.at[r], row_v)

        zero_hist()

        def pass_a(i, c):
            key = _key_of(row_v[pl.ds(i * 16, 16)])
            d = (key >> np.uint32(24)).astype(jnp.int32)
            plsc.addupdate_scatter(hist_v, [d * 16 + lane], ones16)
            return c
        lax.fori_loop(0, NV, pass_a, 0)
        b1, g1 = select_level(np.int32(K))
        need2 = np.int32(K) - g1
        b1u = b1.astype(jnp.uint32)

        zero_hist()

        def pass_b(i, cur):
            key = _key_of(row_v[pl.ds(i * 16, 16)])
            sel = (key >> np.uint32(24)) == b1u
            d2 = ((key >> np.uint32(16)) & np.uint32(0xFF)).astype(jnp.int32)
            plsc.addupdate_scatter(hist_v, [d2 * 16 + lane], ones16, mask=sel)
            seli = jnp.where(sel, 1, 0).astype(jnp.int32)
            pref = jnp.cumsum(seli) - seli
            plsc.store_scatter(cand_v, [cur + pref],
                               plsc.bitcast(key, jnp.int32), mask=sel)
            return cur + jnp.sum(seli)
        c1 = lax.fori_loop(0, NV, pass_b, np.int32(0))
        b2, g2 = select_level(need2)
        need3 = need2 - g2
        b2u = b2.astype(jnp.uint32)

        zero_hist()
        nc1 = (c1 + 15) >> 4

        def pass_c(i, c):
            kv = plsc.bitcast(cand_v[pl.ds(i * 16, 16)], jnp.uint32)
            valid = (i * 16 + lane) < c1
            m = valid & (((kv >> np.uint32(16)) & np.uint32(0xFF)) == b2u)
            d3 = ((kv >> np.uint32(8)) & np.uint32(0xFF)).astype(jnp.int32)
            plsc.addupdate_scatter(hist_v, [d3 * 16 + lane], ones16, mask=m)
            return c
        lax.fori_loop(0, nc1, pass_c, 0)
        b3, g3 = select_level(need3)
        need4 = need3 - g3
        b3u = b3.astype(jnp.uint32)

        zero_hist()

        def pass_d(i, c):
            kv = plsc.bitcast(cand_v[pl.ds(i * 16, 16)], jnp.uint32)
            valid = (i * 16 + lane) < c1
            m = (valid
                 & (((kv >> np.uint32(16)) & np.uint32(0xFF)) == b2u)
                 & (((kv >> np.uint32(8)) & np.uint32(0xFF)) == b3u))
            d4 = (kv & np.uint32(0xFF)).astype(jnp.int32)
            plsc.addupdate_scatter(hist_v, [d4 * 16 + lane], ones16, mask=m)
            return c
        lax.fori_loop(0, nc1, pass_d, 0)
        b4, _g4 = select_level(need4)

        ku = ((b1u << np.uint32(24)) | (b2u << np.uint32(16))
              | (b3u.astype(jnp.uint32) << np.uint32(8))
              | b4.astype(jnp.uint32))
        kuv = jnp.full((16,), ku, jnp.uint32)

        def pass_f(i, carry):
            gcur, ecur = carry
            x = row_v[pl.ds(i * 16, 16)]
            key = _key_of(x)
            gt = key > kuv
            eq = key == kuv
            row_v[pl.ds(i * 16, 16)] = jnp.where(gt, x, NEG)
            idxv = i * 16 + lane
            gti = jnp.where(gt, 1, 0).astype(jnp.int32)
            pg = jnp.cumsum(gti) - gti
            plsc.store_scatter(selk_v, [gcur + pg],
                               plsc.bitcast(key, jnp.int32), mask=gt)
            plsc.store_scatter(seli_v, [gcur + pg], idxv, mask=gt)
            eqi = jnp.where(eq, 1, 0).astype(jnp.int32)
            pe = jnp.cumsum(eqi) - eqi
            plsc.store_scatter(cand_v, [ecur + pe], idxv, mask=eq)
            return gcur + jnp.sum(gti), ecur + jnp.sum(eqi)
        gtot, _etot = lax.fori_loop(0, NV, pass_f,
                                    (np.int32(0), np.int32(0)))

        kiv = plsc.bitcast(kuv, jnp.int32)
        ui = kiv ^ jnp.where(kiv < 0, MIN32, jnp.int32(-1))
        xkv = plsc.bitcast(ui, jnp.float32)
        need_f = np.int32(K) - gtot
        jmax = (need_f + 15) >> 4

        def fix(j, c):
            iv = cand_v[pl.ds(j * 16, 16)]
            valid = (j * 16 + lane) < need_f
            plsc.store_scatter(row_v, [iv], xkv, mask=valid)
            pos = gtot + j * 16 + lane
            plsc.store_scatter(selk_v, [pos], kiv, mask=valid)
            plsc.store_scatter(seli_v, [pos], iv, mask=valid)
            return c
        lax.fori_loop(0, jmax, fix, 0)

        def rank_t(t, c):
            kt = plsc.bitcast(selk_v[pl.ds(t * 16, 16)], jnp.uint32)
            it = seli_v[pl.ds(t * 16, 16)]

            def over_s(sv, acc):
                ksv = selk_v[pl.ds(sv * 16, 16)]
                isv = seli_v[pl.ds(sv * 16, 16)]
                for l in range(16):
                    ksu = plsc.bitcast(
                        jnp.full((16,), ksv[l], jnp.int32), jnp.uint32)
                    iv = jnp.full((16,), isv[l], jnp.int32)
                    m = (ksu > kt) | ((ksu == kt) & (iv < it))
                    acc = acc + jnp.where(m, 1, 0).astype(jnp.int32)
                return acc
            rk = lax.fori_loop(0, 16, over_s, zeros16)
            plsc.store_scatter(oidx_v, [rk], it)
            return c
        lax.fori_loop(0, 16, rank_t, 0)

        pltpu.sync_copy(row_v, masked_hbm.at[r])
        pltpu.sync_copy(oidx_v, idx_hbm.at[r])

    def row_loop(i, c):
        do_row(wid * 4 + i)
        return c
    lax.fori_loop(0, 4, row_loop, 0)


def kernel(scores, k):
    mesh = plsc.VectorSubcoreMesh(core_axis_name="c", subcore_axis_name="s")
    f = pl.kernel(
        _body,
        out_type=(
            jax.ShapeDtypeStruct((B, N), jnp.float32),
            jax.ShapeDtypeStruct((B, K), jnp.int32),
        ),
        mesh=mesh,
        compiler_params=pltpu.CompilerParams(needs_layout_passes=False),
        scratch_types=[
            pltpu.VMEM((N,), jnp.float32),
            pltpu.VMEM((N + 16,), jnp.int32),
            pltpu.VMEM((4096,), jnp.int32),
            pltpu.VMEM((256,), jnp.int32),
            pltpu.VMEM((272,), jnp.int32),
            pltpu.VMEM((272,), jnp.int32),
            pltpu.VMEM((256,), jnp.int32),
        ],
    )
    masked, idx = f(scores)
    return masked, idx
